# baseline (speedup 1.0000x reference)
import functools
import math

import jax
import jax.numpy as jnp
from jax import lax
from jax.experimental import pallas as pl
from jax.experimental.pallas import tpu as pltpu
from jax.experimental.pallas import tpu_sc as plsc

NC = 2
NS = 16
NW = NC * NS
LANES = 16
SPLIT = 96


@functools.partial(jax.jit, static_argnums=(2, 3, 4))
def _lookup(xf, table2, b, l, d):
    bat_w = b // NW
    mesh = plsc.VectorSubcoreMesh(core_axis_name="c", subcore_axis_name="s")
    n_grp = l // LANES
    tail = l - n_grp * LANES

    @functools.partial(
        pl.kernel,
        mesh=mesh,
        out_type=jax.ShapeDtypeStruct((b, l, d), jnp.float32),
        scratch_types=[
            pltpu.VMEM((bat_w, l), jnp.int32),
            pltpu.VMEM((l,), jnp.int32),
            pltpu.VMEM((l, 2 * d), jnp.float32),
            pltpu.VMEM((l, d), jnp.float32),
            pltpu.SemaphoreType.DMA,
        ],
    )
    def k(x_hbm, table_hbm, out_hbm, x_v, pair_v, g, stage, sem):
        wid = lax.axis_index("s") * NC + lax.axis_index("c")
        base = wid * bat_w
        pltpu.sync_copy(x_hbm.at[wid], x_v)

        def sel_row(r, ps):
            for cc in range(d // LANES):
                left = g[r, pl.ds(cc * LANES, LANES)]
                right = g[r, pl.ds(d + cc * LANES, LANES)]
                stage[r, pl.ds(cc * LANES, LANES)] = (
                    left * 8.0 + (right - left) * ps)

        def chunk_body(j, carry):
            for o in range(0, l - LANES + 1, LANES):
                pair_v[pl.ds(o, LANES)] = (
                    x_v[j, pl.ds(o, LANES)] >> 1)
            if tail:
                pair_v[pl.ds(l - LANES, LANES)] = (
                    x_v[j, pl.ds(l - LANES, LANES)] >> 1)

            ca = pltpu.async_copy(
                table_hbm.at[pair_v.at[pl.ds(0, SPLIT)]],
                g.at[pl.ds(0, SPLIT)], sem)
            cb = pltpu.async_copy(
                table_hbm.at[pair_v.at[pl.ds(SPLIT, l - SPLIT)]],
                g.at[pl.ds(SPLIT, l - SPLIT)], sem)
            ca.wait()
            cb.wait()

            def grp_body(gi, c2):
                o = gi * LANES
                pv = ((x_v[j, pl.ds(o, LANES)] & 1)
                      .astype(jnp.float32) * 8.0)
                for rr in range(LANES):
                    sel_row(o + rr, jnp.broadcast_to(pv[rr], (LANES,)))
                return c2

            lax.fori_loop(0, n_grp, grp_body, 0, unroll=False)
            if tail:
                pv = ((x_v[j, pl.ds(l - LANES, LANES)] & 1)
                      .astype(jnp.float32) * 8.0)
                for rr in range(LANES - tail, LANES):
                    sel_row(l - LANES + rr,
                            jnp.broadcast_to(pv[rr], (LANES,)))

            pltpu.sync_copy(stage, out_hbm.at[base + j])
            return carry

        lax.fori_loop(0, bat_w, chunk_body, 0)

    return k(xf, table2)


def kernel(x, table):
    b, l = x.shape
    _, d = table.shape
    xf = x.astype(jnp.int32).reshape(NW, b // NW, l)
    table2 = table.reshape(-1, 2 * d)
    return _lookup(xf, table2, b, l, d)


# speedup vs baseline: 1.5648x; 1.5648x over previous
import functools
import math

import jax
import jax.numpy as jnp
from jax import lax
from jax.experimental import pallas as pl
from jax.experimental.pallas import tpu as pltpu
from jax.experimental.pallas import tpu_sc as plsc

NC = 2
NS = 16
NW = NC * NS
LANES = 16
SPLIT = 96
NBUF = 4


@functools.partial(jax.jit, static_argnums=(2, 3, 4))
def _lookup(xf, table, b, l, d):
    bat_w = b // NW
    n_grp = l // LANES
    tail = l - n_grp * LANES
    mesh = plsc.VectorSubcoreMesh(core_axis_name="c", subcore_axis_name="s")

    @functools.partial(
        pl.kernel,
        mesh=mesh,
        out_type=jax.ShapeDtypeStruct((b * l, d), jnp.float32),
        compiler_params=pltpu.CompilerParams(use_tc_tiling_on_sc=False),
        scratch_types=(
            [pltpu.VMEM((bat_w * l,), jnp.int32)]
            + [pltpu.VMEM((l, d), jnp.float32) for _ in range(NBUF)]
            + [pltpu.SemaphoreType.DMA for _ in range(NBUF)]
        ),
    )
    def k(x_hbm, table_hbm, out_hbm, x_v, *rest):
        g = rest[:NBUF]
        sems = rest[NBUF:2 * NBUF]
        wid = lax.axis_index("s") * NC + lax.axis_index("c")
        base = wid * bat_w
        pltpu.sync_copy(x_hbm.at[wid], x_v)

        def start_gather(j, bi):
            off = pl.multiple_of(j * l, 8)
            pltpu.async_copy(
                table_hbm.at[x_v.at[pl.ds(off, SPLIT)]],
                g[bi].at[pl.ds(0, SPLIT)], sems[bi])
            off2 = pl.multiple_of(j * l + SPLIT, 8)
            pltpu.async_copy(
                table_hbm.at[x_v.at[pl.ds(off2, l - SPLIT)]],
                g[bi].at[pl.ds(SPLIT, l - SPLIT)], sems[bi])

        def wait_buf(bi):
            pltpu.make_async_copy(g[bi], out_hbm.at[pl.ds(0, l)], sems[bi]).wait()

        def scale(bi):
            def grp_body(gi, c2):
                for rr in range(LANES):
                    for cc in range(d // LANES):
                        sl = (gi * LANES + rr, pl.ds(cc * LANES, LANES))
                        g[bi][sl] = g[bi][sl] * 8.0
                return c2

            lax.fori_loop(0, n_grp, grp_body, 0)
            if tail:
                for rr in range(tail):
                    for cc in range(d // LANES):
                        sl = (n_grp * LANES + rr, pl.ds(cc * LANES, LANES))
                        g[bi][sl] = g[bi][sl] * 8.0

        start_gather(0, 0)

        def group_body(g4, carry):
            for bi in range(NBUF):
                j = g4 * NBUF + bi
                nbi = (bi + 1) % NBUF
                wait_buf(bi)
                if bi == NBUF - 1:
                    @pl.when(g4 < (bat_w // NBUF) - 1)
                    def _():
                        wait_buf(nbi)
                        start_gather(j + 1, nbi)
                else:
                    @pl.when(g4 >= 1)
                    def _():
                        wait_buf(nbi)
                        start_gather(j + 1, nbi)

                    @pl.when(g4 == 0)
                    def _():
                        start_gather(j + 1, nbi)
                scale(bi)
                pltpu.async_copy(g[bi], out_hbm.at[pl.ds(pl.multiple_of((base + j) * l, 8), l)], sems[bi])
            return carry

        lax.fori_loop(0, bat_w // NBUF, group_body, 0)
        for bi in range(NBUF):
            wait_buf(bi)

    return k(xf, table)


def kernel(x, table):
    b, l = x.shape
    _, d = table.shape
    xf = x.astype(jnp.int32).reshape(NW, (b // NW) * l)
    return _lookup(xf, table, b, l, d).reshape(b, l, d)
